# Initial kernel scaffold; baseline (speedup 1.0000x reference)
import jax
import jax.numpy as jnp
from jax.experimental import pallas as pl


def kernel(words, raw_weight, mask):
    raise NotImplementedError("write your pallas kernel here")


# speedup vs baseline: 1.7945x; 1.7945x over previous
import functools

import jax
import jax.numpy as jnp
from jax import lax
from jax.experimental import pallas as pl
from jax.experimental.pallas import tpu as pltpu
from jax.experimental.pallas import tpu_sc as plsc

VOCAB = 1000000
DIM = 64
NC = 2
NS = 16
NW = NC * NS
LANES = 16


def _make_kernel(n_total: int, chunk: int):
    per_w = n_total // NW
    n_chunks = per_w // chunk
    mesh = plsc.VectorSubcoreMesh(
        core_axis_name="c", subcore_axis_name="s",
        num_cores=NC, num_subcores=NS,
    )

    @functools.partial(
        pl.kernel,
        mesh=mesh,
        compiler_params=pltpu.CompilerParams(use_tc_tiling_on_sc=False),
        out_type=jax.ShapeDtypeStruct((n_total, DIM), jnp.float32),
        scratch_types=[
            pltpu.VMEM((chunk,), jnp.int32),
            pltpu.VMEM((chunk, DIM), jnp.float32),
            pltpu.VMEM((chunk,), jnp.float32),
            pltpu.SemaphoreType.DMA,
            pltpu.SemaphoreType.DMA,
        ],
    )
    def k(words_hbm, table_hbm, mask_hbm, out_hbm, idx_v, rows_v, maskv_v,
          sem_r, sem_m):
        wid = lax.axis_index("s") * NC + lax.axis_index("c")
        base = wid * per_w

        def do_chunk(g, carry):
            off = base + g * chunk
            pltpu.sync_copy(words_hbm.at[pl.ds(off, chunk)], idx_v)
            cp_r = pltpu.async_copy(table_hbm.at[idx_v], rows_v, sem_r)
            cp_m = pltpu.async_copy(mask_hbm.at[idx_v], maskv_v, sem_m)
            cp_r.wait()
            cp_m.wait()

            def rowgrp(g16, c):
                mvec = maskv_v[pl.ds(g16 * LANES, LANES)]
                for r in range(LANES):
                    i = g16 * LANES + r
                    m = mvec[r]
                    for j in range(DIM // LANES):
                        sl = pl.ds(j * LANES, LANES)
                        rows_v[i, sl] = rows_v[i, sl] * m
                return c

            lax.fori_loop(0, chunk // LANES, rowgrp, 0)
            pltpu.sync_copy(rows_v, out_hbm.at[pl.ds(off, chunk)])
            return carry

        lax.fori_loop(0, n_chunks, do_chunk, 0)

    return k


@jax.jit
def kernel(words, raw_weight, mask):
    b, l = words.shape
    n_total = b * l
    flat_words = words.reshape(n_total).astype(jnp.int32)
    flat_mask = mask.reshape(-1)
    k = _make_kernel(n_total, 800)
    out = k(flat_words, raw_weight, flat_mask)
    return out.reshape(b, l, DIM)
